# baseline (speedup 1.0000x reference)
import functools

import jax
import jax.numpy as jnp
from jax import lax
from jax.experimental import pallas as pl
from jax.experimental.pallas import tpu as pltpu
from jax.experimental.pallas import tpu_sc as plsc

VOCAB = 1000000
B = 4096
L = 200
EMB = 32

NC = 2
NS = 16
NW = NC * NS

BB = B // NW
CT = EMB // 8
LSTRIDE = CT * NW * 8 * BB
TILE_ELS = 8 * BB
BLK = CT * TILE_ELS
NLOADS = BB * EMB // 16


def _sc_gather(idx_lb, table):
    mesh = plsc.VectorSubcoreMesh(
        core_axis_name="c", subcore_axis_name="s",
        num_cores=NC, num_subcores=NS)

    @functools.partial(
        pl.kernel,
        mesh=mesh,
        out_type=jax.ShapeDtypeStruct((L * LSTRIDE,), jnp.float32),
        scratch_types=[
            pltpu.VMEM((L, BB), jnp.int32),
            pltpu.VMEM((4, BB, EMB), jnp.float32),
            pltpu.VMEM((4, BLK), jnp.float32),
            pltpu.SemaphoreType.DMA,
            pltpu.SemaphoreType.DMA,
        ],
        compiler_params=pltpu.CompilerParams(
            use_tc_tiling_on_sc=False, needs_layout_passes=False),
    )
    def k(idx_hbm, table_hbm, out_hbm, idx_v, rows_v, tile_v, gsem, wsem):
        wid = lax.axis_index("s") * NC + lax.axis_index("c")
        b0 = wid * BB
        pltpu.sync_copy(idx_hbm.at[:, pl.ds(b0, BB)], idx_v)

        iota = lax.broadcasted_iota(jnp.int32, (16,), 0)
        dst_base = [iota * BB, (iota + 16) * BB]

        def fire_gather(l, p):
            pltpu.async_copy(
                table_hbm.at[idx_v.at[l]], rows_v.at[p], gsem)

        def wait_gather(p):
            pltpu.make_async_copy(
                table_hbm.at[idx_v.at[0]], rows_v.at[p], gsem).wait()

        def transpose(p, q):
            for mg in range(0, NLOADS, 8):
                vals = [
                    rows_v[p, (mg + i) // 2, pl.ds(16 * ((mg + i) % 2), 16)]
                    for i in range(8)
                ]
                for i in range(8):
                    m = mg + i
                    plsc.store_scatter(
                        tile_v.at[q], [dst_base[m % 2] + (m // 2)], vals[i])

        def fire_write(l, q):
            base = l * LSTRIDE + wid * TILE_ELS
            for ct in range(CT):
                pltpu.async_copy(
                    tile_v.at[q].at[pl.ds(ct * TILE_ELS, TILE_ELS)],
                    out_hbm.at[pl.ds(base + ct * NW * TILE_ELS, TILE_ELS)],
                    wsem)

        def wait_write(q):
            for ct in range(CT):
                pltpu.make_async_copy(
                    tile_v.at[q].at[pl.ds(ct * TILE_ELS, TILE_ELS)],
                    out_hbm.at[pl.ds(ct * TILE_ELS, TILE_ELS)],
                    wsem).wait()

        def step(l, p, q):
            @pl.when(l + 3 < L)
            def _():
                fire_gather(l + 3, (p + 3) % 4)

            wait_gather(p)

            @pl.when(l >= 4)
            def _():
                wait_write(q)

            transpose(p, q)
            fire_write(l, q)

        fire_gather(0, 0)
        fire_gather(1, 1)
        fire_gather(2, 2)

        def body(u, carry):
            for r in range(4):
                step(4 * u + r, r, r)
            return carry

        lax.fori_loop(0, L // 4, body, 0)
        for q in range(4):
            wait_write(q)

    return k(idx_lb, table)


def kernel(inputs, table):
    idx_lb = jnp.swapaxes(inputs, 0, 1).astype(jnp.int32)
    flat = _sc_gather(idx_lb, table)
    out5 = flat.reshape(L, CT, NW, 8, BB)
    return jnp.transpose(out5, (2, 4, 0, 1, 3)).reshape(B, L, EMB)


# speedup vs baseline: 1.0176x; 1.0176x over previous
import functools

import jax
import jax.numpy as jnp
from jax import lax
from jax.experimental import pallas as pl
from jax.experimental.pallas import tpu as pltpu
from jax.experimental.pallas import tpu_sc as plsc

B = 4096
L = 200
EMB = 32

NC = 2
NS = 16
NW = NC * NS

BB = B // NW
K = 4
NBATCH = L // K
NPAIR = NBATCH // 2


def _sc_embedding_lookup(idx_lb, table):
    mesh = plsc.VectorSubcoreMesh(
        core_axis_name="c", subcore_axis_name="s",
        num_cores=NC, num_subcores=NS)

    @functools.partial(
        pl.kernel,
        mesh=mesh,
        out_type=jax.ShapeDtypeStruct((L, B, EMB), jnp.float32),
        scratch_types=[
            pltpu.VMEM((L, BB), jnp.int32),
            pltpu.VMEM((2, K, BB, EMB), jnp.float32),
            pltpu.SemaphoreType.DMA,
            pltpu.SemaphoreType.DMA,
        ],
        compiler_params=pltpu.CompilerParams(use_tc_tiling_on_sc=False),
    )
    def k(idx_hbm, table_hbm, out_hbm, idx_v, rows_v, gsem, wsem):
        wid = lax.axis_index("s") * NC + lax.axis_index("c")
        b0 = wid * BB
        pltpu.sync_copy(idx_hbm.at[:, pl.ds(b0, BB)], idx_v)

        def fire(t, p):
            return [
                pltpu.async_copy(
                    table_hbm.at[idx_v.at[K * t + j]],
                    rows_v.at[p].at[j],
                    gsem)
                for j in range(K)
            ]

        def writeback(t, p):
            pltpu.async_copy(
                rows_v.at[p],
                out_hbm.at[pl.ds(K * t, K), pl.ds(b0, BB)],
                wsem)

        def wait_writeback(p):
            pltpu.make_async_copy(
                rows_v.at[p],
                out_hbm.at[pl.ds(0, K), pl.ds(b0, BB)],
                wsem).wait()

        def body(t, carry):
            a = 2 * t

            @pl.when(t > 0)
            def _():
                wait_writeback(0)

            ga = fire(a, 0)

            @pl.when(t > 0)
            def _():
                wait_writeback(1)

            gb = fire(a + 1, 1)
            for cp in ga:
                cp.wait()
            writeback(a, 0)
            for cp in gb:
                cp.wait()
            writeback(a + 1, 1)
            return carry

        lax.fori_loop(0, NPAIR, body, 0)
        wait_writeback(0)
        wait_writeback(1)

    return k(idx_lb, table)


def kernel(inputs, table):
    idx_lb = jnp.swapaxes(inputs, 0, 1).astype(jnp.int32)
    out = _sc_embedding_lookup(idx_lb, table)
    return jnp.transpose(out, (1, 0, 2))
